# baseline (speedup 1.0000x reference)
import functools

import jax
import jax.numpy as jnp
from jax import lax
from jax.experimental import pallas as pl
from jax.experimental.pallas import tpu as pltpu
from jax.experimental.pallas import tpu_sc as plsc

NUM_CORES = 2
NUM_SUBCORES = 16
NUM_WORKERS = NUM_CORES * NUM_SUBCORES
GB = 8


def _make_gather(batch: int, seq: int, dim: int):
    assert batch % (NUM_WORKERS * GB * 2) == 0
    b_per_w = batch // NUM_WORKERS
    num_pairs = b_per_w // (2 * GB)

    mesh = plsc.VectorSubcoreMesh(
        core_axis_name="c", subcore_axis_name="s",
        num_cores=NUM_CORES, num_subcores=NUM_SUBCORES)

    @functools.partial(
        pl.kernel,
        out_type=jax.ShapeDtypeStruct((batch, seq, dim), jnp.float32),
        mesh=mesh,
        compiler_params=pltpu.CompilerParams(use_tc_tiling_on_sc=False),
        scratch_types=[
            pltpu.VMEM((b_per_w, seq), jnp.int32),
            pltpu.VMEM((GB, seq, dim), jnp.float32),
            pltpu.VMEM((GB, seq, dim), jnp.float32),
            pltpu.SemaphoreType.DMA,
            pltpu.SemaphoreType.DMA,
            pltpu.SemaphoreType.DMA,
        ],
    )
    def gather_kernel(idx_hbm, table_hbm, out_hbm, idx_v, rows_a, rows_b,
                      gsem_a, gsem_b, osem):
        wid = lax.axis_index("s") * NUM_CORES + lax.axis_index("c")
        b_base = wid * b_per_w

        pltpu.sync_copy(idx_hbm.at[pl.ds(b_base, b_per_w)], idx_v)

        def fire(group, buf, sem):
            for j in range(GB):
                pltpu.async_copy(
                    table_hbm.at[idx_v.at[group * GB + j]],
                    buf.at[j], sem)

        def drain_gathers(buf, sem):
            pltpu.make_async_copy(
                out_hbm.at[pl.ds(0, GB)], buf, sem).wait()

        def scatter(group, buf):
            pltpu.async_copy(
                buf, out_hbm.at[pl.ds(b_base + group * GB, GB)],
                osem).wait()

        fire(0, rows_a, gsem_a)

        def body(q, _):
            a = 2 * q
            fire(a + 1, rows_b, gsem_b)
            drain_gathers(rows_a, gsem_a)
            scatter(a, rows_a)

            @pl.when(q < num_pairs - 1)
            def _():
                fire(a + 2, rows_a, gsem_a)

            drain_gathers(rows_b, gsem_b)
            scatter(a + 1, rows_b)
            return 0

        lax.fori_loop(0, num_pairs, body, 0)

    return gather_kernel


def kernel(x, embedding):
    b, s = x.shape
    return _make_gather(b, s, embedding.shape[1])(
        x.astype(jnp.int32), embedding)
